# baseline (speedup 1.0000x reference)
import jax
import jax.numpy as jnp
from jax import lax
from jax.experimental import pallas as pl
from jax.experimental.pallas import tpu as pltpu
from jax.experimental.pallas import tpu_sc as plsc

NC = 2
NS = 16
NW = NC * NS

CHUNK = 128


def _gather_body(idx_hbm, tab_hbm, out_hbm, idx_v, rows_v, gsem):
    c = lax.axis_index("c")
    s = lax.axis_index("s")
    wid = s * NC + c
    n_chunks = idx_v.shape[0]
    pltpu.sync_copy(idx_hbm.at[wid], idx_v)
    base = wid * (n_chunks * CHUNK)

    def step(j, carry):
        pltpu.async_copy(tab_hbm.at[idx_v.at[j]], rows_v, gsem).wait()
        pltpu.sync_copy(rows_v, out_hbm.at[pl.ds(base + j * CHUNK, CHUNK)])
        return carry

    lax.fori_loop(0, n_chunks, step, 0)


def kernel(indices, embeddings):
    B, H = indices.shape
    V, two, D = embeddings.shape
    N = B * H
    row = two * D
    assert N % (NW * CHUNK) == 0
    n_chunks = N // (NW * CHUNK)

    idx3 = indices.reshape(NW, n_chunks, CHUNK)
    tab2 = embeddings.reshape(V, row)

    mesh = plsc.VectorSubcoreMesh(core_axis_name="c", subcore_axis_name="s",
                                  num_cores=NC, num_subcores=NS)
    out = pl.kernel(
        _gather_body,
        out_type=jax.ShapeDtypeStruct((N, row), jnp.float32),
        mesh=mesh,
        scratch_types=[
            pltpu.VMEM((n_chunks, CHUNK), jnp.int32),
            pltpu.VMEM((CHUNK, row), jnp.float32),
            pltpu.SemaphoreType.DMA,
        ],
    )(idx3, tab2)
    return out.reshape(B, H, two, D)


# speedup vs baseline: 1.0426x; 1.0426x over previous
import jax
import jax.numpy as jnp
from jax import lax
from jax.experimental import pallas as pl
from jax.experimental.pallas import tpu as pltpu
from jax.experimental.pallas import tpu_sc as plsc

NC = 2
NS = 16
NW = NC * NS

CHUNK = 128
NBUF = 4


def _gather_body(idx_hbm, tab_hbm, out_hbm, idx_v, rows_v, gsem, wsem):
    c = lax.axis_index("c")
    s = lax.axis_index("s")
    wid = s * NC + c
    n_chunks = idx_v.shape[0]
    n_groups = n_chunks // NBUF
    pltpu.sync_copy(idx_hbm.at[wid], idx_v)
    base = wid * (n_chunks * CHUNK)

    def gather_desc(j, b):
        return pltpu.make_async_copy(
            tab_hbm.at[idx_v.at[j]], rows_v.at[b], gsem.at[b])

    def write_desc(j, b):
        return pltpu.make_async_copy(
            rows_v.at[b], out_hbm.at[pl.ds(base + j * CHUNK, CHUNK)],
            wsem.at[b])

    gather_desc(0, 0).start()
    gather_desc(1, 1).start()
    for b in range(NBUF):
        gather_desc(b, b).wait()
        write_desc(b, b).start()
        if b >= 2:
            write_desc(b - 2, (b + 2) % NBUF).wait()
        gather_desc(b + 2, (b + 2) % NBUF).start()

    def group(g, carry):
        j0 = g * NBUF
        for b in range(NBUF):
            gather_desc(j0 + b, b).wait()
            write_desc(j0 + b, b).start()
            write_desc(j0 + b - 2, (b + 2) % NBUF).wait()
            gather_desc(j0 + b + 2, (b + 2) % NBUF).start()
        return carry

    lax.fori_loop(1, n_groups - 1, group, 0)

    j0 = (n_groups - 1) * NBUF
    for b in range(NBUF):
        gather_desc(j0 + b, b).wait()
        write_desc(j0 + b, b).start()
        if b < 2:
            write_desc(j0 + b - 2, (b + 2) % NBUF).wait()
            gather_desc(j0 + b + 2, (b + 2) % NBUF).start()
    for b in range(NBUF):
        write_desc(j0 + b, b).wait()


def kernel(indices, embeddings):
    B, H = indices.shape
    V, two, D = embeddings.shape
    N = B * H
    row = two * D
    assert N % (NW * CHUNK) == 0
    n_chunks = N // (NW * CHUNK)
    assert n_chunks % NBUF == 0 and n_chunks >= 3 * NBUF

    idx3 = indices.reshape(NW, n_chunks, CHUNK)
    tab2 = embeddings.reshape(V, row)

    mesh = plsc.VectorSubcoreMesh(core_axis_name="c", subcore_axis_name="s",
                                  num_cores=NC, num_subcores=NS)
    out = pl.kernel(
        _gather_body,
        out_type=jax.ShapeDtypeStruct((N, row), jnp.float32),
        mesh=mesh,
        scratch_types=[
            pltpu.VMEM((n_chunks, CHUNK), jnp.int32),
            pltpu.VMEM((NBUF, CHUNK, row), jnp.float32),
            pltpu.SemaphoreType.DMA((NBUF,)),
            pltpu.SemaphoreType.DMA((NBUF,)),
        ],
    )(idx3, tab2)
    return out.reshape(B, H, two, D)


# speedup vs baseline: 1.0654x; 1.0218x over previous
import jax
import jax.numpy as jnp
from jax import lax
from jax.experimental import pallas as pl
from jax.experimental.pallas import tpu as pltpu
from jax.experimental.pallas import tpu_sc as plsc

NC = 2
NS = 16
NW = NC * NS

CHUNK = 128
L = 16


def _gather_body(idx_hbm, tab_hbm, out_hbm, idx_v, grows, trans, gsem, wsem):
    c = lax.axis_index("c")
    s = lax.axis_index("s")
    w = s * NC + c
    n_blocks = idx_v.shape[0]
    n_tc = out_hbm.shape[3]
    pltpu.sync_copy(idx_hbm.at[w], idx_v)

    lane = lax.iota(jnp.int32, L)
    bi_vecs = [lane + (g * L) for g in range(8)]

    def g_desc(k, b):
        return pltpu.make_async_copy(
            tab_hbm.at[idx_v.at[k]], grows.at[b], gsem.at[b])

    def w_desc(k, b):
        blk = w * n_blocks + k
        h = blk // n_tc
        tc = blk % n_tc
        return pltpu.make_async_copy(
            trans.at[b], out_hbm.at[h, :, :, tc], wsem.at[b])

    def transpose_block(b):
        src = grows.at[b]
        dst = trans.at[b]

        def col_group(dcg, carry):
            ci = dcg // 8
            tri = dcg % 8
            for di in range(8):
                dc = dcg * 8 + di
                dc_vec = jnp.full((L,), 0, jnp.int32) + dc
                for g in range(8):
                    vals = plsc.load_gather(src, [bi_vecs[g], dc_vec])
                    dst[ci, tri, di, pl.ds(g * L, L)] = vals
            return carry

        lax.fori_loop(0, 16, col_group, 0)

    g_desc(0, 0).start()
    for k in range(2):
        b = k % 2
        g_desc(k, b).wait()
        if k + 1 < n_blocks:
            g_desc(k + 1, (k + 1) % 2).start()
        transpose_block(b)
        w_desc(k, b).start()

    def pair(p, carry):
        k0 = p * 2
        for b in range(2):
            k = k0 + b
            g_desc(k, b).wait()
            g_desc(k + 1, (b + 1) % 2).start()
            w_desc(k - 2, b).wait()
            transpose_block(b)
            w_desc(k, b).start()
        return carry

    lax.fori_loop(1, n_blocks // 2 - 1, pair, 0)

    k0 = n_blocks - 2
    for k in range(k0, n_blocks):
        b = k % 2
        g_desc(k, b).wait()
        if k + 1 < n_blocks:
            g_desc(k + 1, (k + 1) % 2).start()
        w_desc(k - 2, b).wait()
        transpose_block(b)
        w_desc(k, b).start()
    for k in range(k0, n_blocks):
        w_desc(k, k % 2).wait()


def kernel(indices, embeddings):
    B, H = indices.shape
    V, two, D = embeddings.shape
    N = B * H
    row = two * D
    assert row == 128 and B % 128 == 0 and N % (NW * CHUNK) == 0
    n_blocks = N // (NW * CHUNK)

    idx3 = indices.T.reshape(NW, n_blocks, CHUNK)
    tab2 = embeddings.reshape(V, row)

    mesh = plsc.VectorSubcoreMesh(core_axis_name="c", subcore_axis_name="s",
                                  num_cores=NC, num_subcores=NS)
    out = pl.kernel(
        _gather_body,
        out_type=jax.ShapeDtypeStruct((H, two, 8, B // 128, 8, 128),
                                      jnp.float32),
        mesh=mesh,
        compiler_params=pltpu.CompilerParams(needs_layout_passes=False),
        scratch_types=[
            pltpu.VMEM((n_blocks, CHUNK), jnp.int32),
            pltpu.VMEM((2, CHUNK, row), jnp.float32),
            pltpu.VMEM((2, two, 8, 8, 128), jnp.float32),
            pltpu.SemaphoreType.DMA((2,)),
            pltpu.SemaphoreType.DMA((2,)),
        ],
    )(idx3, tab2)
    out = out.transpose(3, 5, 0, 1, 2, 4)
    return out.reshape(B, H, two, D)
